# Initial kernel scaffold; baseline (speedup 1.0000x reference)
import jax
import jax.numpy as jnp
from jax.experimental import pallas as pl


def kernel(x, table):
    raise NotImplementedError("write your pallas kernel here")


# speedup vs baseline: 1.5734x; 1.5734x over previous
import functools

import jax
import jax.numpy as jnp
from jax import lax
from jax.experimental import pallas as pl
from jax.experimental.pallas import tpu as pltpu
from jax.experimental.pallas import tpu_sc as plsc

_BATCH = 16384
_HIST = 50
_DIM = 64
_B = _BATCH * _HIST

_NC = 2
_NS = 16
_NW = _NC * _NS
_BPW = _B // _NW
_CHUNK = 128
_NCHUNK = _BPW // _CHUNK

_mesh = plsc.VectorSubcoreMesh(core_axis_name="c", subcore_axis_name="s")


@functools.partial(
    pl.kernel,
    mesh=_mesh,
    compiler_params=pltpu.CompilerParams(use_tc_tiling_on_sc=False),
    out_type=jax.ShapeDtypeStruct((_B, _DIM), jnp.float32),
    scratch_types=[
        pltpu.VMEM((_CHUNK,), jnp.int32),
        pltpu.VMEM((_CHUNK, _DIM), jnp.float32),
        pltpu.SemaphoreType.DMA,
    ],
)
def _gather_kernel(idx_hbm, table_hbm, out_hbm, idx_v, rows_v, sem):
    wid = lax.axis_index("s") * _NC + lax.axis_index("c")
    base = wid * _BPW

    def body(g, carry):
        off = base + g * _CHUNK
        pltpu.sync_copy(idx_hbm.at[pl.ds(off, _CHUNK)], idx_v)
        pltpu.async_copy(table_hbm.at[idx_v], rows_v, sem).wait()
        pltpu.sync_copy(rows_v, out_hbm.at[pl.ds(off, _CHUNK)])
        return carry

    lax.fori_loop(0, _NCHUNK, body, 0)


def kernel(x, table):
    idx = x.reshape(-1).astype(jnp.int32)
    out = _gather_kernel(idx, table)
    return out.reshape(_BATCH, _HIST, _DIM)


# speedup vs baseline: 1.8719x; 1.1898x over previous
import functools

import jax
import jax.numpy as jnp
from jax import lax
from jax.experimental import pallas as pl
from jax.experimental.pallas import tpu as pltpu
from jax.experimental.pallas import tpu_sc as plsc

_BATCH = 16384
_HIST = 50
_DIM = 64
_B = _BATCH * _HIST

_NC = 2
_NS = 16
_NW = _NC * _NS
_BPW = _B // _NW
_CHUNK = 512
_NCHUNK = _BPW // _CHUNK

_mesh = plsc.VectorSubcoreMesh(core_axis_name="c", subcore_axis_name="s")


@functools.partial(
    pl.kernel,
    mesh=_mesh,
    compiler_params=pltpu.CompilerParams(use_tc_tiling_on_sc=False),
    out_type=jax.ShapeDtypeStruct((_B, _DIM), jnp.float32),
    scratch_types=[
        pltpu.VMEM((_BPW,), jnp.int32),
        pltpu.VMEM((_CHUNK, _DIM), jnp.float32),
        pltpu.VMEM((_CHUNK, _DIM), jnp.float32),
        pltpu.SemaphoreType.DMA,
        pltpu.SemaphoreType.DMA,
        pltpu.SemaphoreType.DMA,
        pltpu.SemaphoreType.DMA,
    ],
)
def _gather_kernel(idx_hbm, table_hbm, out_hbm, idx_all, rows0, rows1,
                   gsem0, gsem1, wsem0, wsem1):
    wid = lax.axis_index("s") * _NC + lax.axis_index("c")
    base = wid * _BPW
    pltpu.sync_copy(idx_hbm.at[pl.ds(base, _BPW)], idx_all)

    rows = (rows0, rows1)
    gsem = (gsem0, gsem1)
    wsem = (wsem0, wsem1)

    def fire_gather(g, b):
        src = table_hbm.at[idx_all.at[pl.ds(g * _CHUNK, _CHUNK)]]
        return pltpu.async_copy(src, rows[b], gsem[b])

    def start_write(g, b):
        dst = out_hbm.at[pl.ds(base + g * _CHUNK, _CHUNK)]
        pltpu.async_copy(rows[b], dst, wsem[b])

    def wait_write(b):
        dst = out_hbm.at[pl.ds(base, _CHUNK)]
        pltpu.make_async_copy(rows[b], dst, wsem[b]).wait()

    def pair_body(i, carry):
        for b in range(2):
            g = 2 * i + b

            @pl.when(g >= 2)
            def _():
                wait_write(b)

            fire_gather(g, b).wait()
            start_write(g, b)
        return carry

    lax.fori_loop(0, _NCHUNK // 2, pair_body, 0)
    wait_write(0)
    wait_write(1)


def kernel(x, table):
    idx = x.reshape(-1).astype(jnp.int32)
    out = _gather_kernel(idx, table)
    return out.reshape(_BATCH, _HIST, _DIM)


# speedup vs baseline: 1.8753x; 1.0018x over previous
import functools

import jax
import jax.numpy as jnp
from jax import lax
from jax.experimental import pallas as pl
from jax.experimental.pallas import tpu as pltpu
from jax.experimental.pallas import tpu_sc as plsc

_BATCH = 16384
_HIST = 50
_DIM = 64
_B = _BATCH * _HIST

_NC = 2
_NS = 16
_NW = _NC * _NS
_BPW = _B // _NW
_CHUNK = 128
_NCHUNK = _BPW // _CHUNK
_NB = 8
_LOOKAHEAD = 6

_mesh = plsc.VectorSubcoreMesh(core_axis_name="c", subcore_axis_name="s")


@functools.partial(
    pl.kernel,
    mesh=_mesh,
    compiler_params=pltpu.CompilerParams(use_tc_tiling_on_sc=False),
    out_type=jax.ShapeDtypeStruct((_B, _DIM), jnp.float32),
    scratch_types=[
        pltpu.VMEM((_BPW,), jnp.int32),
        pltpu.VMEM((_NB, _CHUNK, _DIM), jnp.float32),
    ]
    + [pltpu.SemaphoreType.DMA] * (2 * _NB),
)
def _gather_kernel(idx_hbm, table_hbm, out_hbm, idx_all, rows, *sems):
    gsem = sems[:_NB]
    wsem = sems[_NB:]
    wid = lax.axis_index("s") * _NC + lax.axis_index("c")
    base = wid * _BPW
    pltpu.sync_copy(idx_hbm.at[pl.ds(base, _BPW)], idx_all)

    def fire_gather(g, b):
        src = table_hbm.at[idx_all.at[pl.ds(g * _CHUNK, _CHUNK)]]
        pltpu.async_copy(src, rows.at[b], gsem[b])

    def wait_gather(b):
        src = table_hbm.at[idx_all.at[pl.ds(0, _CHUNK)]]
        pltpu.make_async_copy(src, rows.at[b], gsem[b]).wait()

    def start_write(g, b):
        dst = out_hbm.at[pl.ds(base + g * _CHUNK, _CHUNK)]
        pltpu.async_copy(rows.at[b], dst, wsem[b])

    def wait_write(b):
        dst = out_hbm.at[pl.ds(base, _CHUNK)]
        pltpu.make_async_copy(rows.at[b], dst, wsem[b]).wait()

    for g in range(_LOOKAHEAD):
        fire_gather(g, g)

    def group_body(sg, carry):
        for b in range(_NB):
            g = sg * _NB + b
            wait_gather(b)
            start_write(g, b)
            f = g + _LOOKAHEAD
            bf = (b + _LOOKAHEAD) % _NB

            @pl.when((f >= _NB) & (f < _NCHUNK))
            def _():
                wait_write(bf)

            @pl.when(f < _NCHUNK)
            def _():
                fire_gather(f, bf)

        return carry

    lax.fori_loop(0, _NCHUNK // _NB, group_body, 0)
    for b in range(_NB):
        wait_write(b)


def kernel(x, table):
    idx = x.reshape(-1).astype(jnp.int32)
    out = _gather_kernel(idx, table)
    return out.reshape(_BATCH, _HIST, _DIM)


# speedup vs baseline: 1.9590x; 1.0446x over previous
import functools

import jax
import jax.numpy as jnp
from jax import lax
from jax.experimental import pallas as pl
from jax.experimental.pallas import tpu as pltpu
from jax.experimental.pallas import tpu_sc as plsc

_BATCH = 16384
_HIST = 50
_DIM = 64
_B = _BATCH * _HIST

_NC = 2
_NS = 16
_NW = _NC * _NS
_BPW = _B // _NW
_CHUNK = 128
_NCHUNK = _BPW // _CHUNK
_NB = 8
_LOOKAHEAD = 6

_mesh = plsc.VectorSubcoreMesh(core_axis_name="c", subcore_axis_name="s")


@functools.partial(
    pl.kernel,
    mesh=_mesh,
    compiler_params=pltpu.CompilerParams(use_tc_tiling_on_sc=False),
    out_type=jax.ShapeDtypeStruct((_B, _DIM), jnp.float32),
    scratch_types=[
        pltpu.VMEM((_BPW,), jnp.int32),
        pltpu.VMEM((_NB, _CHUNK, _DIM), jnp.float32),
    ]
    + [pltpu.SemaphoreType.DMA] * (2 * _NB),
)
def _gather_kernel(idx_hbm, table_hbm, out_hbm, idx_all, rows, *sems):
    gsem = sems[:_NB]
    wsem = sems[_NB:]
    wid = lax.axis_index("s") * _NC + lax.axis_index("c")
    base = wid * _BPW
    pltpu.sync_copy(idx_hbm.at[pl.ds(base, _BPW)], idx_all)

    def fire_gather(g, b):
        src = table_hbm.at[idx_all.at[pl.ds(g * _CHUNK, _CHUNK)]]
        pltpu.async_copy(src, rows.at[b], gsem[b])

    def wait_gather(b):
        src = table_hbm.at[idx_all.at[pl.ds(0, _CHUNK)]]
        pltpu.make_async_copy(src, rows.at[b], gsem[b]).wait()

    def start_write(g, b):
        dst = out_hbm.at[pl.ds(base + g * _CHUNK, _CHUNK)]
        pltpu.async_copy(rows.at[b], dst, wsem[b])

    def wait_write(b):
        dst = out_hbm.at[pl.ds(base, _CHUNK)]
        pltpu.make_async_copy(rows.at[b], dst, wsem[b]).wait()

    for g in range(_LOOKAHEAD):
        fire_gather(g, g)

    def group_body(sg, carry):
        for b in range(_NB):
            g = sg * _NB + b
            wait_gather(b)
            start_write(g, b)
            f = g + _LOOKAHEAD
            bf = (b + _LOOKAHEAD) % _NB

            @pl.when((f >= _NB) & (f < _NCHUNK))
            def _():
                wait_write(bf)

            @pl.when(f < _NCHUNK)
            def _():
                fire_gather(f, bf)

        return carry

    lax.fori_loop(0, _NCHUNK // _NB, group_body, 0)
    for b in range(_NB):
        wait_write(b)


def kernel(x, table):
    idx = x.T.reshape(-1).astype(jnp.int32)
    out = _gather_kernel(idx, table)
    return out.reshape(_HIST, _BATCH, _DIM).transpose(1, 0, 2)
